# baseline (speedup 1.0000x reference)
import jax
import jax.numpy as jnp
from jax import lax
from jax.experimental import pallas as pl
from jax.experimental.pallas import tpu as pltpu
from jax.experimental.pallas import tpu_sc as plsc

DIM = 64
GW = 128
PADW = 128
NBUF = 2


def _make_kernel(B, NC, NS):
    NW = NC * NS
    n_groups = B // GW
    n_per_w = n_groups // NW

    mesh = plsc.VectorSubcoreMesh(
        core_axis_name="c", subcore_axis_name="s",
        num_cores=NC, num_subcores=NS)

    @pl.kernel(
        out_type=jax.ShapeDtypeStruct((n_groups // 32 * 8, 32, 1024),
                                      jnp.float32),
        mesh=mesh,
        scratch_types=[
            pltpu.VMEM((n_per_w, GW), jnp.int32),
            pltpu.VMEM((NBUF, GW, PADW), jnp.float32),
            pltpu.VMEM((NBUF, 8, 1024), jnp.float32),
            pltpu.SemaphoreType.DMA((NBUF,)),
            pltpu.SemaphoreType.DMA((NBUF,)),
        ],
        compiler_params=pltpu.CompilerParams(use_tc_tiling_on_sc=False,
                                             needs_layout_passes=False,
                                             disable_bounds_checks=True),
    )
    def k(table_hbm, idx_hbm, out_hbm, idx_v, g_v, t_v, sem_g, sem_w):
        wid = lax.axis_index("s") * NC + lax.axis_index("c")
        g0 = wid * n_per_w

        pltpu.sync_copy(idx_hbm.at[pl.ds(g0, n_per_w)], idx_v)

        def fire_gather(gl, b):
            pltpu.async_copy(table_hbm.at[idx_v.at[gl]], g_v.at[b],
                             sem_g.at[b])

        def wait_gather(gl, b):
            pltpu.make_async_copy(table_hbm.at[idx_v.at[gl]], g_v.at[b],
                                  sem_g.at[b]).wait()

        def out_pos(gl):
            g = g0 + gl
            ti = g >> 8
            rem = g & 255
            j = rem >> 3
            r = rem & 7
            s = ti * 8 + r
            return s * 8, j

        def start_wb(gl, b):
            s8, j = out_pos(gl)
            pltpu.async_copy(t_v.at[b], out_hbm.at[pl.ds(s8, 8), j],
                             sem_w.at[b])

        def wait_wb(gl, b):
            s8, j = out_pos(gl)
            pltpu.make_async_copy(t_v.at[b], out_hbm.at[pl.ds(s8, 8), j],
                                  sem_w.at[b]).wait()

        iota = lax.iota(jnp.int32, 16)
        row_ids = [iota + v * 16 for v in range(8)]

        def transpose(b):
            gb = g_v.at[b]

            def tbody(d8, carry):
                tb = t_v.at[b].at[d8]
                for du in range(0, 8, 2):
                    vals = []
                    for du2 in range(2):
                        d = d8 * 8 + du + du2
                        col = jnp.full((16,), d, jnp.int32)
                        for v in range(8):
                            vals.append(plsc.load_gather(gb,
                                                         [row_ids[v], col]))
                    i = 0
                    for du2 in range(2):
                        for v in range(8):
                            tb[pl.ds((du + du2) * 128 + v * 16, 16)] = vals[i]
                            i += 1
                return carry

            lax.fori_loop(0, 8, tbody, 0)

        fire_gather(0, 0)

        def body(i, carry):
            for b in range(NBUF):
                gl = i * NBUF + b

                @pl.when(gl + 1 < n_per_w)
                def _():
                    fire_gather(gl + 1, 1 - b)

                wait_gather(gl, b)

                @pl.when(gl >= NBUF)
                def _():
                    wait_wb(gl - NBUF, b)

                transpose(b)
                start_wb(gl, b)
            return carry

        lax.fori_loop(0, n_per_w // NBUF, body, 0)

        wait_wb(n_per_w - 2, 0)
        wait_wb(n_per_w - 1, 1)

    return k


def kernel(input_ids, table):
    BATCH, SEQ = input_ids.shape
    VOCAB = table.shape[0]
    B = BATCH * SEQ
    info = plsc.get_sparse_core_info()
    NC, NS = info.num_cores, info.num_subcores

    ids4 = input_ids.T.reshape(SEQ // 8, 8, BATCH // 128, 128)
    ids4 = ids4.transpose(0, 2, 1, 3)
    idx2d = ids4.reshape(B // GW, GW)

    table128 = jnp.pad(table, ((0, 0), (0, PADW - DIM)))

    k = _make_kernel(B, NC, NS)
    out = k(table128, idx2d)

    out5 = out.reshape(SEQ, 8, BATCH // 128, 8, 128)
    t1 = out5.transpose(0, 1, 3, 2, 4).reshape(SEQ, DIM, BATCH)
    return t1.transpose(2, 0, 1)


# speedup vs baseline: 1.5809x; 1.5809x over previous
import jax
import jax.numpy as jnp
from jax import lax
from jax.experimental import pallas as pl
from jax.experimental.pallas import tpu as pltpu
from jax.experimental.pallas import tpu_sc as plsc

DIM = 64
GW = 128
PADW = 128
NBUF = 2


def _make_kernel(B, NC, NS):
    NW = NC * NS
    n_groups = B // GW
    n_per_w = n_groups // NW

    mesh = plsc.VectorSubcoreMesh(
        core_axis_name="c", subcore_axis_name="s",
        num_cores=NC, num_subcores=NS)

    @pl.kernel(
        out_type=jax.ShapeDtypeStruct((n_groups // 32 * 8, 32, 8, 128),
                                      jnp.float32),
        mesh=mesh,
        scratch_types=[
            pltpu.VMEM((n_per_w, GW), jnp.int32),
            pltpu.VMEM((NBUF, GW, PADW), jnp.float32),
            pltpu.VMEM((NBUF, 8, 8, 128), jnp.float32),
            pltpu.SemaphoreType.DMA((NBUF,)),
            pltpu.SemaphoreType.DMA((NBUF,)),
        ],
        compiler_params=pltpu.CompilerParams(use_tc_tiling_on_sc=False,
                                             needs_layout_passes=False,
                                             disable_bounds_checks=True),
    )
    def k(table_hbm, idx_hbm, out_hbm, idx_v, g_v, t_v, sem_g, sem_w):
        wid = lax.axis_index("s") * NC + lax.axis_index("c")
        g0 = wid * n_per_w

        pltpu.sync_copy(idx_hbm.at[pl.ds(g0, n_per_w)], idx_v)

        def fire_gather(gl, b):
            pltpu.async_copy(table_hbm.at[idx_v.at[gl]], g_v.at[b],
                             sem_g.at[b])

        def wait_gather(gl, b):
            pltpu.make_async_copy(table_hbm.at[idx_v.at[gl]], g_v.at[b],
                                  sem_g.at[b]).wait()

        def out_pos(gl):
            g = g0 + gl
            ti = g >> 8
            rem = g & 255
            j = rem >> 3
            r = rem & 7
            s = ti * 8 + r
            return s * 8, j

        def start_wb(gl, b):
            s8, j = out_pos(gl)
            pltpu.async_copy(t_v.at[b], out_hbm.at[pl.ds(s8, 8), j],
                             sem_w.at[b])

        def wait_wb(gl, b):
            s8, j = out_pos(gl)
            pltpu.make_async_copy(t_v.at[b], out_hbm.at[pl.ds(s8, 8), j],
                                  sem_w.at[b]).wait()

        iota = lax.iota(jnp.int32, 16)
        row_ids = [iota + v * 16 for v in range(8)]

        def transpose(b):
            gb = g_v.at[b]
            tb = t_v.at[b]

            def tbody(d4, carry):
                for du in range(4):
                    d = d4 * 4 + du
                    dcol = (d + iota) & 63
                    dg = dcol >> 3
                    dr = dcol & 7
                    for v in range(8):
                        val = plsc.load_gather(gb, [row_ids[v], dcol])
                        plsc.store_scatter(tb, [dg, dr, row_ids[v]], val)
                return carry

            lax.fori_loop(0, 16, tbody, 0)

        fire_gather(0, 0)

        def body(i, carry):
            for b in range(NBUF):
                gl = i * NBUF + b

                @pl.when(gl + 1 < n_per_w)
                def _():
                    fire_gather(gl + 1, 1 - b)

                wait_gather(gl, b)

                @pl.when(gl >= NBUF)
                def _():
                    wait_wb(gl - NBUF, b)

                transpose(b)
                start_wb(gl, b)
            return carry

        lax.fori_loop(0, n_per_w // NBUF, body, 0)

        wait_wb(n_per_w - 2, 0)
        wait_wb(n_per_w - 1, 1)

    return k


def kernel(input_ids, table):
    BATCH, SEQ = input_ids.shape
    VOCAB = table.shape[0]
    B = BATCH * SEQ
    info = plsc.get_sparse_core_info()
    NC, NS = info.num_cores, info.num_subcores

    ids4 = input_ids.T.reshape(SEQ // 8, 8, BATCH // 128, 128)
    ids4 = ids4.transpose(0, 2, 1, 3)
    idx2d = ids4.reshape(B // GW, GW)

    table128 = jnp.pad(table, ((0, 0), (0, PADW - DIM)))

    k = _make_kernel(B, NC, NS)
    out = k(table128, idx2d)

    out5 = out.reshape(SEQ, 8, BATCH // 128, 8, 128)
    t1 = out5.transpose(0, 1, 3, 2, 4).reshape(SEQ, DIM, BATCH)
    return t1.transpose(2, 0, 1)


# speedup vs baseline: 1.6167x; 1.0226x over previous
import jax
import jax.numpy as jnp
from jax import lax
from jax.experimental import pallas as pl
from jax.experimental.pallas import tpu as pltpu
from jax.experimental.pallas import tpu_sc as plsc

DIM = 64
GW = 128
PRW = 128
NBUF = 2


def _make_repack(VOCAB, NC, NS):
    NW = NC * NS
    n_blocks = VOCAB // GW
    per_w = (n_blocks + NW - 1) // NW

    mesh = plsc.VectorSubcoreMesh(
        core_axis_name="c", subcore_axis_name="s",
        num_cores=NC, num_subcores=NS)

    T = (per_w + NBUF - 1) // NBUF * NBUF

    @pl.kernel(
        out_type=jax.ShapeDtypeStruct((VOCAB // 2, PRW), jnp.float32),
        mesh=mesh,
        scratch_types=[
            pltpu.VMEM((NBUF, DIM, GW), jnp.float32),
            pltpu.VMEM((NBUF, DIM, GW), jnp.float32),
            pltpu.SemaphoreType.DMA((NBUF,)),
            pltpu.SemaphoreType.DMA((NBUF,)),
        ],
        compiler_params=pltpu.CompilerParams(use_tc_tiling_on_sc=True,
                                             needs_layout_passes=False,
                                             disable_bounds_checks=True),
    )
    def k(tabT_hbm, tail_hbm, out_hbm, s_v, d_v, sem_r, sem_w):
        wid = lax.axis_index("s") * NC + lax.axis_index("c")
        c0 = wid * per_w

        def blk(cl):
            return jnp.minimum(c0 + cl, n_blocks - 1)

        def fire_read(cl, b):
            off = pl.multiple_of(blk(cl) * GW, GW)
            pltpu.async_copy(tabT_hbm.at[:, pl.ds(off, GW)], s_v.at[b],
                             sem_r.at[b])

        def wait_read(cl, b):
            off = pl.multiple_of(blk(cl) * GW, GW)
            pltpu.make_async_copy(tabT_hbm.at[:, pl.ds(off, GW)], s_v.at[b],
                                  sem_r.at[b]).wait()

        def start_wb(cl, b):
            pltpu.async_copy(d_v.at[b],
                             out_hbm.at[pl.ds(blk(cl) * DIM, DIM)],
                             sem_w.at[b])

        def wait_wb(cl, b):
            pltpu.make_async_copy(d_v.at[b],
                                  out_hbm.at[pl.ds(blk(cl) * DIM, DIM)],
                                  sem_w.at[b]).wait()

        iota = lax.iota(jnp.int32, 16)
        row_ids = [iota + v * 16 for v in range(8)]

        def transpose(src, dst):
            def tbody(d4, carry):
                for du in range(4):
                    d0 = d4 * 4 + du
                    dcol = (d0 + iota) & 63
                    for v in range(8):
                        val = plsc.load_gather(src, [dcol, row_ids[v]])
                        flat = row_ids[v] * 64 + dcol
                        plsc.store_scatter(dst, [flat >> 7, flat & 127], val)
                return carry

            lax.fori_loop(0, 16, tbody, 0)

        fire_read(0, 0)

        def body(i, carry):
            for b in range(NBUF):
                cl = i * NBUF + b

                @pl.when(cl + 1 < T)
                def _():
                    fire_read(cl + 1, 1 - b)

                wait_read(cl, b)

                @pl.when(cl >= NBUF)
                def _():
                    wait_wb(cl - NBUF, b)

                transpose(s_v.at[b], d_v.at[b])
                start_wb(cl, b)
            return carry

        lax.fori_loop(0, T // NBUF, body, 0)

        for j in range(NBUF):
            cl = T - NBUF + j
            wait_wb(cl, cl % NBUF)

        @pl.when(wid == NW - 1)
        def _():
            pltpu.async_copy(tail_hbm, s_v.at[0], sem_r.at[0])
            pltpu.make_async_copy(tail_hbm, s_v.at[0], sem_r.at[0]).wait()
            transpose(s_v.at[0], d_v.at[0])
            pltpu.async_copy(d_v.at[0],
                             out_hbm.at[pl.ds(VOCAB // 2 - DIM, DIM)],
                             sem_w.at[0])
            pltpu.make_async_copy(d_v.at[0],
                                  out_hbm.at[pl.ds(VOCAB // 2 - DIM, DIM)],
                                  sem_w.at[0]).wait()

    return k


def _make_lookup(B, NC, NS):
    NW = NC * NS
    n_groups = B // GW
    n_per_w = n_groups // NW

    mesh = plsc.VectorSubcoreMesh(
        core_axis_name="c", subcore_axis_name="s",
        num_cores=NC, num_subcores=NS)

    @pl.kernel(
        out_type=jax.ShapeDtypeStruct((n_groups // 32 * 8, 32, 8, 128),
                                      jnp.float32),
        mesh=mesh,
        scratch_types=[
            pltpu.VMEM((n_per_w, GW), jnp.int32),
            pltpu.VMEM((n_per_w, GW), jnp.int32),
            pltpu.VMEM((NBUF, GW, PRW), jnp.float32),
            pltpu.VMEM((NBUF, 8, 8, 128), jnp.float32),
            pltpu.SemaphoreType.DMA((NBUF,)),
            pltpu.SemaphoreType.DMA((NBUF,)),
        ],
        compiler_params=pltpu.CompilerParams(use_tc_tiling_on_sc=True,
                                             needs_layout_passes=False,
                                             disable_bounds_checks=True),
    )
    def k(tab2_hbm, idx_hbm, out_hbm, idx_v, pair_v, g_v, t_v, sem_g, sem_w):
        wid = lax.axis_index("s") * NC + lax.axis_index("c")
        g0 = wid * n_per_w

        pltpu.sync_copy(idx_hbm.at[pl.ds(g0, n_per_w)], idx_v)

        def pbody(gl, carry):
            src = idx_v.at[gl]
            dst = pair_v.at[gl]
            for v in range(8):
                dst[pl.ds(v * 16, 16)] = src[pl.ds(v * 16, 16)] >> 1
            return carry

        lax.fori_loop(0, n_per_w, pbody, 0)

        def fire_gather(gl, b):
            pltpu.async_copy(tab2_hbm.at[pair_v.at[gl]], g_v.at[b],
                             sem_g.at[b])

        def wait_gather(gl, b):
            pltpu.make_async_copy(tab2_hbm.at[pair_v.at[gl]], g_v.at[b],
                                  sem_g.at[b]).wait()

        def out_pos(gl):
            g = g0 + gl
            ti = g >> 8
            rem = g & 255
            j = rem >> 3
            r = rem & 7
            s = ti * 8 + r
            return s * 8, j

        def start_wb(gl, b):
            s8, j = out_pos(gl)
            pltpu.async_copy(t_v.at[b], out_hbm.at[pl.ds(s8, 8), j],
                             sem_w.at[b])

        def wait_wb(gl, b):
            s8, j = out_pos(gl)
            pltpu.make_async_copy(t_v.at[b], out_hbm.at[pl.ds(s8, 8), j],
                                  sem_w.at[b]).wait()

        iota = lax.iota(jnp.int32, 16)
        row_ids = [iota + v * 16 for v in range(8)]

        def transpose(gl, b):
            gb = g_v.at[b]
            tb = t_v.at[b]
            idxrow = idx_v.at[gl]
            par = [(idxrow[pl.ds(v * 16, 16)] & 1) << 6 for v in range(8)]

            def tbody(d4, carry):
                for du in range(4):
                    d0 = d4 * 4 + du
                    dcol = (d0 + iota) & 63
                    dg = dcol >> 3
                    dr = dcol & 7
                    for v in range(8):
                        val = plsc.load_gather(gb, [row_ids[v],
                                                    dcol + par[v]])
                        plsc.store_scatter(tb, [dg, dr, row_ids[v]], val)
                return carry

            lax.fori_loop(0, 16, tbody, 0)

        fire_gather(0, 0)

        def body(i, carry):
            for b in range(NBUF):
                gl = i * NBUF + b

                @pl.when(gl + 1 < n_per_w)
                def _():
                    fire_gather(gl + 1, 1 - b)

                wait_gather(gl, b)

                @pl.when(gl >= NBUF)
                def _():
                    wait_wb(gl - NBUF, b)

                transpose(gl, b)
                start_wb(gl, b)
            return carry

        lax.fori_loop(0, n_per_w // NBUF, body, 0)

        wait_wb(n_per_w - 2, 0)
        wait_wb(n_per_w - 1, 1)

    return k


def kernel(input_ids, table):
    BATCH, SEQ = input_ids.shape
    VOCAB = table.shape[0]
    B = BATCH * SEQ
    info = plsc.get_sparse_core_info()
    NC, NS = info.num_cores, info.num_subcores

    ids4 = input_ids.T.reshape(SEQ // 8, 8, BATCH // 128, 128)
    ids4 = ids4.transpose(0, 2, 1, 3)
    idx2d = ids4.reshape(B // GW, GW)

    tabT = table.T
    tail = table[VOCAB - GW:].T

    repack = _make_repack(VOCAB, NC, NS)
    tab2 = repack(tabT, tail)

    lookup = _make_lookup(B, NC, NS)
    out = lookup(tab2, idx2d)

    out5 = out.reshape(SEQ, 8, BATCH // 128, 8, 128)
    t1 = out5.transpose(0, 1, 3, 2, 4).reshape(SEQ, DIM, BATCH)
    return t1.transpose(2, 0, 1)
